# baseline (speedup 1.0000x reference)
import functools

import jax
import jax.numpy as jnp
from jax import lax
from jax.experimental import pallas as pl
from jax.experimental.pallas import tpu as pltpu
from jax.experimental.pallas import tpu_sc as plsc

EMB = 64
BATCH = 16384
NUM_CORES = 2
NUM_SUBCORES = 16
NUM_WORKERS = NUM_CORES * NUM_SUBCORES
B_PER_W = BATCH // NUM_WORKERS
CHUNK = 128
N_CHUNKS = B_PER_W // CHUNK
LANES = 16


@functools.partial(
    pl.kernel,
    out_type=jax.ShapeDtypeStruct((BATCH, EMB), jnp.float32),
    mesh=plsc.VectorSubcoreMesh(core_axis_name="c", subcore_axis_name="s"),
    compiler_params=pltpu.CompilerParams(use_tc_tiling_on_sc=False),
    scratch_types=[
        pltpu.VMEM((N_CHUNKS, CHUNK), jnp.int32),
        pltpu.VMEM((N_CHUNKS, CHUNK), jnp.int32),
        pltpu.VMEM((B_PER_W, EMB), jnp.float32),
        pltpu.VMEM((B_PER_W, EMB), jnp.float32),
        pltpu.SemaphoreType.DMA,
    ],
)
def _gmf_sc(uid_hbm, mid_hbm, ut_hbm, it_hbm, out_hbm,
            uidx, midx, urows, mrows, sem):
    wid = lax.axis_index("s") * NUM_CORES + lax.axis_index("c")
    pltpu.sync_copy(uid_hbm.at[wid], uidx)
    pltpu.sync_copy(mid_hbm.at[wid], midx)
    copies = []
    for j in range(N_CHUNKS):
        copies.append(pltpu.async_copy(
            ut_hbm.at[uidx.at[j]], urows.at[pl.ds(j * CHUNK, CHUNK)], sem))
        copies.append(pltpu.async_copy(
            it_hbm.at[midx.at[j]], mrows.at[pl.ds(j * CHUNK, CHUNK)], sem))
    for c in copies:
        c.wait()

    def body(i, carry):
        for c in range(EMB // LANES):
            sl = pl.ds(c * LANES, LANES)
            urows[i, sl] = urows[i, sl] * mrows[i, sl]
        return carry

    lax.fori_loop(0, B_PER_W, body, 0)
    pltpu.sync_copy(urows, out_hbm.at[pl.ds(wid * B_PER_W, B_PER_W)])


def kernel(user_ids, movie_ids, user_table, item_table):
    uid = user_ids.astype(jnp.int32).reshape(NUM_WORKERS, N_CHUNKS, CHUNK)
    mid = movie_ids.astype(jnp.int32).reshape(NUM_WORKERS, N_CHUNKS, CHUNK)
    return _gmf_sc(uid, mid, user_table, item_table)


# speedup vs baseline: 1.4684x; 1.4684x over previous
import functools

import jax
import jax.numpy as jnp
from jax import lax
from jax.experimental import pallas as pl
from jax.experimental.pallas import tpu as pltpu
from jax.experimental.pallas import tpu_sc as plsc

EMB = 64
BATCH = 16384
NUM_CORES = 2
NUM_SUBCORES = 16
NUM_WORKERS = NUM_CORES * NUM_SUBCORES
B_PER_W = BATCH // NUM_WORKERS
LANES = 16
W_ELEMS = B_PER_W * EMB
GROUP = 16
N_GROUPS = B_PER_W // GROUP
TILE_ROWS = 8


@functools.partial(
    pl.kernel,
    out_type=jax.ShapeDtypeStruct((BATCH * EMB,), jnp.float32),
    mesh=plsc.VectorSubcoreMesh(core_axis_name="c", subcore_axis_name="s"),
    scratch_types=[
        pltpu.VMEM((B_PER_W,), jnp.int32),
        pltpu.VMEM((B_PER_W,), jnp.int32),
        pltpu.VMEM((2, GROUP, TILE_ROWS, EMB), jnp.float32),
        pltpu.VMEM((2, GROUP, TILE_ROWS, EMB), jnp.float32),
        pltpu.VMEM((W_ELEMS,), jnp.float32),
        pltpu.SemaphoreType.DMA,
        pltpu.SemaphoreType.DMA,
    ],
)
def _gmf_sc(uid_hbm, mid_hbm, ut_hbm, it_hbm, out_hbm,
            uidx, midx, utiles, mtiles, prows, sem_a, sem_b):
    wid = lax.axis_index("s") * NUM_CORES + lax.axis_index("c")
    base = wid * B_PER_W
    pltpu.sync_copy(uid_hbm.at[pl.ds(base, B_PER_W)], uidx)
    pltpu.sync_copy(mid_hbm.at[pl.ds(base, B_PER_W)], midx)

    def fire(g, slot, sem):
        goff = pl.multiple_of(g * GROUP, GROUP)
        tu = uidx[pl.ds(goff, GROUP)] & -TILE_ROWS
        tm = midx[pl.ds(goff, GROUP)] & -TILE_ROWS
        for j in range(GROUP):
            src_u = pl.ds(pl.multiple_of(tu[j], TILE_ROWS), TILE_ROWS)
            src_m = pl.ds(pl.multiple_of(tm[j], TILE_ROWS), TILE_ROWS)
            pltpu.async_copy(ut_hbm.at[src_u], utiles.at[slot, j], sem)
            pltpu.async_copy(it_hbm.at[src_m], mtiles.at[slot, j], sem)

    def drain(sem):
        pltpu.make_async_copy(out_hbm.at[pl.ds(0, W_ELEMS)], prows, sem).wait()

    def process(g, slot):
        goff = pl.multiple_of(g * GROUP, GROUP)
        su = uidx[pl.ds(goff, GROUP)] & (TILE_ROWS - 1)
        sm = midx[pl.ds(goff, GROUP)] & (TILE_ROWS - 1)
        for j in range(GROUP):
            for c in range(EMB // LANES):
                sl = pl.ds(c * LANES, LANES)
                p = utiles[slot, j, su[j], sl] * mtiles[slot, j, sm[j], sl]
                prows[pl.ds((goff + j) * EMB + c * LANES, LANES)] = p

    fire(0, 0, sem_a)

    def body(k, carry):
        g = pl.multiple_of(k * 2, 2)
        fire(g + 1, 1, sem_b)
        drain(sem_a)
        process(g, 0)
        fire(g + 2, 0, sem_a)
        drain(sem_b)
        process(g + 1, 1)
        return carry

    lax.fori_loop(0, N_GROUPS // 2 - 1, body, 0)
    fire(N_GROUPS - 1, 1, sem_b)
    drain(sem_a)
    process(N_GROUPS - 2, 0)
    drain(sem_b)
    process(N_GROUPS - 1, 1)

    pltpu.sync_copy(prows, out_hbm.at[pl.ds(base * EMB, W_ELEMS)])


def kernel(user_ids, movie_ids, user_table, item_table):
    uid = user_ids.astype(jnp.int32)
    mid = movie_ids.astype(jnp.int32)
    flat = _gmf_sc(uid, mid, user_table, item_table)
    return flat.reshape(BATCH, EMB)
